# baseline (speedup 1.0000x reference)
import functools
import math

import jax
import jax.numpy as jnp
from jax import lax
from jax.experimental import pallas as pl
from jax.experimental.pallas import tpu as pltpu
from jax.experimental.pallas import tpu_sc as plsc

_LANES = 16
_CH = 8
_NBUF = 8


@functools.cache
def _build(BATCH, SEQ, D, scale):
    info = plsc.get_sparse_core_info()
    NC, NS = info.num_cores, info.num_subcores
    NW = NC * NS
    B = BATCH * SEQ
    assert B % NW == 0
    b_per_w = B // NW
    assert SEQ % b_per_w == 0
    w_per_row = SEQ // b_per_w
    CH, NBUF = _CH, _NBUF
    assert b_per_w % (CH * NBUF) == 0
    n_chunks = b_per_w // CH
    groups = CH * (D // _LANES)
    cshift = (D // _LANES).bit_length() - 1
    assert D // _LANES == 1 << cshift
    mesh = plsc.VectorSubcoreMesh(core_axis_name="c", subcore_axis_name="s")

    @functools.partial(
        pl.kernel,
        mesh=mesh,
        out_type=jax.ShapeDtypeStruct((BATCH, SEQ, D), jnp.float32),
        scratch_types=[
            pltpu.VMEM((b_per_w,), jnp.int32),
            pltpu.VMEM((NBUF, CH, D), jnp.float32),
        ]
        + [pltpu.SemaphoreType.DMA] * (2 * NBUF),
    )
    def emb(idx_hbm, table_hbm, out_hbm, idx_v, rows_v, *sems):
        gsem, ssem = sems[:NBUF], sems[NBUF:]
        wid = lax.axis_index("s") * NC + lax.axis_index("c")
        row = wid // w_per_row
        col = (wid % w_per_row) * b_per_w
        pltpu.sync_copy(idx_hbm.at[row, pl.ds(col, b_per_w)], idx_v)

        def gather(j, b):
            return pltpu.make_async_copy(
                table_hbm.at[idx_v.at[pl.ds(j * CH, CH)]], rows_v.at[b], gsem[b]
            )

        def scatter(j, b):
            return pltpu.make_async_copy(
                rows_v.at[b], out_hbm.at[row, pl.ds(col + j * CH, CH)], ssem[b]
            )

        for j in range(NBUF - 1):
            gather(j, j).start()

        @pl.loop(0, n_chunks, step=NBUF)
        def _(outer):
            for b in range(NBUF):
                j = outer + b
                nb = (b - 1) % NBUF
                nxt = j + NBUF - 1

                @pl.when(nxt < n_chunks)
                def _():
                    @pl.when(nxt >= NBUF)
                    def _():
                        scatter(nxt - NBUF, nb).wait()

                    gather(nxt, nb).start()

                gather(j, b).wait()

                @plsc.parallel_loop(0, groups, unroll=4)
                def _(i):
                    r = lax.shift_right_logical(i, cshift)
                    c = (i & (D // _LANES - 1)) * _LANES
                    rows_v[b, r, pl.ds(c, _LANES)] = (
                        rows_v[b, r, pl.ds(c, _LANES)] * jnp.float32(scale)
                    )

                scatter(j, b).start()

        for j in range(n_chunks - NBUF, n_chunks):
            scatter(j, j % NBUF).wait()

    return emb


def kernel(x, table):
    BATCH, SEQ = x.shape
    V, D = table.shape
    scale = float(math.sqrt(D))
    return _build(BATCH, SEQ, D, scale)(x.astype(jnp.int32), table)


# speedup vs baseline: 1.1317x; 1.1317x over previous
import functools
import math

import jax
import jax.numpy as jnp
from jax import lax
from jax.experimental import pallas as pl
from jax.experimental.pallas import tpu as pltpu
from jax.experimental.pallas import tpu_sc as plsc

_LANES = 16
_CH = 8
_NBUF = 8


@functools.cache
def _build(BATCH, SEQ, D, scale):
    info = plsc.get_sparse_core_info()
    NC, NS = info.num_cores, info.num_subcores
    NW = NC * NS
    B = BATCH * SEQ
    assert B % NW == 0
    b_per_w = B // NW
    assert SEQ % b_per_w == 0
    w_per_row = SEQ // b_per_w
    CH, NBUF = _CH, _NBUF
    assert b_per_w % (CH * NBUF) == 0
    n_chunks = b_per_w // CH
    groups = CH * (D // _LANES)
    cshift = (D // _LANES).bit_length() - 1
    assert D // _LANES == 1 << cshift
    mesh = plsc.VectorSubcoreMesh(core_axis_name="c", subcore_axis_name="s")

    @functools.partial(
        pl.kernel,
        mesh=mesh,
        out_type=jax.ShapeDtypeStruct((BATCH, SEQ, D), jnp.float32),
        scratch_types=[
            pltpu.VMEM((b_per_w,), jnp.int32),
            pltpu.VMEM((NBUF, CH, D), jnp.float32),
        ]
        + [pltpu.SemaphoreType.DMA] * (2 * NBUF),
    )
    def emb(idx_hbm, table_hbm, out_hbm, idx_v, rows_v, *sems):
        gsem, ssem = sems[:NBUF], sems[NBUF:]
        wid = lax.axis_index("s") * NC + lax.axis_index("c")
        row = wid // w_per_row
        col = (wid % w_per_row) * b_per_w
        pltpu.sync_copy(idx_hbm.at[row, pl.ds(col, b_per_w)], idx_v)

        def gather(j, b):
            return pltpu.make_async_copy(
                table_hbm.at[idx_v.at[pl.ds(j * CH, CH)]], rows_v.at[b], gsem[b]
            )

        def scatter(j, b):
            return pltpu.make_async_copy(
                rows_v.at[b], out_hbm.at[row, pl.ds(col + j * CH, CH)], ssem[b]
            )

        for j in range(NBUF - 1):
            gather(j, j).start()

        @pl.loop(0, n_chunks, step=NBUF)
        def _(outer):
            for b in range(NBUF):
                j = outer + b
                nb = (b - 1) % NBUF
                nxt = j + NBUF - 1

                @pl.when(nxt < n_chunks)
                def _():
                    @pl.when(nxt >= NBUF)
                    def _():
                        scatter(nxt - NBUF, nb).wait()

                    gather(nxt, nb).start()

                gather(j, b).wait()

                @plsc.parallel_loop(0, groups, unroll=16)
                def _(i):
                    r = lax.shift_right_logical(i, cshift)
                    c = (i & (D // _LANES - 1)) * _LANES
                    rows_v[b, r, pl.ds(c, _LANES)] = (
                        rows_v[b, r, pl.ds(c, _LANES)] * jnp.float32(scale)
                    )

                scatter(j, b).start()

        for j in range(n_chunks - NBUF, n_chunks):
            scatter(j, j % NBUF).wait()

    return emb


def kernel(x, table):
    BATCH, SEQ = x.shape
    V, D = table.shape
    scale = float(math.sqrt(D))
    return _build(BATCH, SEQ, D, scale)(x.astype(jnp.int32), table)


# speedup vs baseline: 1.1327x; 1.0008x over previous
import functools
import math

import jax
import jax.numpy as jnp
from jax import lax
from jax.experimental import pallas as pl
from jax.experimental.pallas import tpu as pltpu
from jax.experimental.pallas import tpu_sc as plsc

_LANES = 16
_CH = 8
_NBUF = 8


@functools.cache
def _build(BATCH, SEQ, D, scale):
    info = plsc.get_sparse_core_info()
    NC, NS = info.num_cores, info.num_subcores
    NW = NC * NS
    B = BATCH * SEQ
    assert B % NW == 0
    b_per_w = B // NW
    assert SEQ % b_per_w == 0
    w_per_row = SEQ // b_per_w
    CH, NBUF = _CH, _NBUF
    assert b_per_w % (CH * NBUF) == 0
    n_chunks = b_per_w // CH
    groups = CH * (D // _LANES)
    cshift = (D // _LANES).bit_length() - 1
    assert D // _LANES == 1 << cshift
    mesh = plsc.VectorSubcoreMesh(core_axis_name="c", subcore_axis_name="s")

    @functools.partial(
        pl.kernel,
        mesh=mesh,
        out_type=jax.ShapeDtypeStruct((BATCH, SEQ, D), jnp.float32),
        scratch_types=[
            pltpu.VMEM((b_per_w,), jnp.int32),
            pltpu.VMEM((NBUF, CH, D), jnp.float32),
        ]
        + [pltpu.SemaphoreType.DMA] * (2 * NBUF),
    )
    def emb(idx_hbm, table_hbm, out_hbm, idx_v, rows_v, *sems):
        gsem, ssem = sems[:NBUF], sems[NBUF:]
        wid = lax.axis_index("s") * NC + lax.axis_index("c")
        row = wid // w_per_row
        col = (wid % w_per_row) * b_per_w
        pltpu.sync_copy(idx_hbm.at[row, pl.ds(col, b_per_w)], idx_v)

        def gather(j, b):
            return pltpu.make_async_copy(
                table_hbm.at[idx_v.at[pl.ds(j * CH, CH)]], rows_v.at[b], gsem[b]
            )

        def scatter(j, b):
            return pltpu.make_async_copy(
                rows_v.at[b], out_hbm.at[row, pl.ds(col + j * CH, CH)], ssem[b]
            )

        for j in range(NBUF - 1):
            gather(j, j).start()

        @pl.loop(0, n_chunks, step=NBUF)
        def _(outer):
            for b in range(NBUF):
                j = outer + b
                nb = (b - 1) % NBUF
                nxt = j + NBUF - 1

                @pl.when(nxt < n_chunks)
                def _():
                    @pl.when(nxt >= NBUF)
                    def _():
                        scatter(nxt - NBUF, nb).wait()

                    gather(nxt, nb).start()

                gather(j, b).wait()

                @plsc.parallel_loop(0, CH)
                def _(r):
                    @plsc.parallel_loop(0, D, step=_LANES, unroll=8)
                    def _(c):
                        rows_v[b, r, pl.ds(c, _LANES)] = (
                            rows_v[b, r, pl.ds(c, _LANES)] * jnp.float32(scale)
                        )

                scatter(j, b).start()

        for j in range(n_chunks - NBUF, n_chunks):
            scatter(j, j % NBUF).wait()

    return emb


def kernel(x, table):
    BATCH, SEQ = x.shape
    V, D = table.shape
    scale = float(math.sqrt(D))
    return _build(BATCH, SEQ, D, scale)(x.astype(jnp.int32), table)


# speedup vs baseline: 1.1510x; 1.0162x over previous
import functools
import math

import jax
import jax.numpy as jnp
from jax import lax
from jax.experimental import pallas as pl
from jax.experimental.pallas import tpu as pltpu
from jax.experimental.pallas import tpu_sc as plsc

_LANES = 16
_CH = 8
_NBUF = 8


@functools.cache
def _build(BATCH, SEQ, D, scale):
    info = plsc.get_sparse_core_info()
    NC, NS = info.num_cores, info.num_subcores
    NW = NC * NS
    B = BATCH * SEQ
    assert B % NW == 0
    b_per_w = B // NW
    assert SEQ % b_per_w == 0
    w_per_row = SEQ // b_per_w
    CH, NBUF = _CH, _NBUF
    MASK = NBUF - 1
    assert NBUF & MASK == 0
    assert b_per_w % (CH * NBUF) == 0
    n_chunks = b_per_w // CH
    mesh = plsc.VectorSubcoreMesh(core_axis_name="c", subcore_axis_name="s")

    @functools.partial(
        pl.kernel,
        mesh=mesh,
        out_type=jax.ShapeDtypeStruct((BATCH, SEQ, D), jnp.float32),
        scratch_types=[
            pltpu.VMEM((b_per_w,), jnp.int32),
            pltpu.VMEM((NBUF, CH, D), jnp.float32),
            pltpu.SemaphoreType.DMA((NBUF,)),
            pltpu.SemaphoreType.DMA((NBUF,)),
        ],
    )
    def emb(idx_hbm, table_hbm, out_hbm, idx_v, rows_v, gsem, ssem):
        wid = lax.axis_index("s") * NC + lax.axis_index("c")
        row = wid // w_per_row
        col = (wid % w_per_row) * b_per_w
        pltpu.sync_copy(idx_hbm.at[row, pl.ds(col, b_per_w)], idx_v)

        def gather(j, b):
            return pltpu.make_async_copy(
                table_hbm.at[idx_v.at[pl.ds(j * CH, CH)]],
                rows_v.at[b],
                gsem.at[b],
            )

        def scatter(j, b):
            return pltpu.make_async_copy(
                rows_v.at[b],
                out_hbm.at[row, pl.ds(col + j * CH, CH)],
                ssem.at[b],
            )

        @pl.loop(0, NBUF - 1)
        def _(j):
            gather(j, j & MASK).start()

        @pl.loop(0, n_chunks)
        def _(j):
            b = j & MASK
            pb = (j - 1) & MASK
            nxt = j + NBUF - 1

            @pl.when(nxt < n_chunks)
            def _():
                @pl.when(nxt >= NBUF)
                def _():
                    scatter(nxt - NBUF, pb).wait()

                gather(nxt, pb).start()

            gather(j, b).wait()

            @plsc.parallel_loop(0, CH)
            def _(r):
                @plsc.parallel_loop(0, D, step=_LANES, unroll=8)
                def _(c):
                    rows_v[b, r, pl.ds(c, _LANES)] = (
                        rows_v[b, r, pl.ds(c, _LANES)] * jnp.float32(scale)
                    )

            scatter(j, b).start()

        @pl.loop(n_chunks - NBUF, n_chunks)
        def _(j):
            scatter(j, j & MASK).wait()

    return emb


def kernel(x, table):
    BATCH, SEQ = x.shape
    V, D = table.shape
    scale = float(math.sqrt(D))
    return _build(BATCH, SEQ, D, scale)(x.astype(jnp.int32), table)


# speedup vs baseline: 1.1737x; 1.0197x over previous
import functools
import math

import jax
import jax.numpy as jnp
from jax import lax
from jax.experimental import pallas as pl
from jax.experimental.pallas import tpu as pltpu
from jax.experimental.pallas import tpu_sc as plsc

_LANES = 16
_CH = 8
_NBUF = 15


@functools.cache
def _build(BATCH, SEQ, D, scale):
    info = plsc.get_sparse_core_info()
    NC, NS = info.num_cores, info.num_subcores
    NW = NC * NS
    B = BATCH * SEQ
    assert B % NW == 0
    b_per_w = B // NW
    assert SEQ % b_per_w == 0
    w_per_row = SEQ // b_per_w
    CH, NBUF = _CH, _NBUF
    assert b_per_w % CH == 0
    n_chunks = b_per_w // CH

    def bmod(j):
        if NBUF & (NBUF - 1) == 0:
            return j & (NBUF - 1)
        return lax.rem(j, NBUF)
    mesh = plsc.VectorSubcoreMesh(core_axis_name="c", subcore_axis_name="s")

    @functools.partial(
        pl.kernel,
        mesh=mesh,
        out_type=jax.ShapeDtypeStruct((BATCH, SEQ, D), jnp.float32),
        scratch_types=[
            pltpu.VMEM((b_per_w,), jnp.int32),
            pltpu.VMEM((NBUF, CH, D), jnp.float32),
            pltpu.SemaphoreType.DMA((NBUF,)),
            pltpu.SemaphoreType.DMA((NBUF,)),
        ],
    )
    def emb(idx_hbm, table_hbm, out_hbm, idx_v, rows_v, gsem, ssem):
        wid = lax.axis_index("s") * NC + lax.axis_index("c")
        row = wid // w_per_row
        col = (wid % w_per_row) * b_per_w
        pltpu.sync_copy(idx_hbm.at[row, pl.ds(col, b_per_w)], idx_v)

        def gather(j, b):
            return pltpu.make_async_copy(
                table_hbm.at[idx_v.at[pl.ds(j * CH, CH)]],
                rows_v.at[b],
                gsem.at[b],
            )

        def scatter(j, b):
            return pltpu.make_async_copy(
                rows_v.at[b],
                out_hbm.at[row, pl.ds(col + j * CH, CH)],
                ssem.at[b],
            )

        @pl.loop(0, NBUF - 1)
        def _(j):
            gather(j, bmod(j)).start()

        @pl.loop(0, n_chunks)
        def _(j):
            b = bmod(j)
            pb = bmod(j + NBUF - 1)
            nxt = j + NBUF - 1

            @pl.when(nxt < n_chunks)
            def _():
                @pl.when(nxt >= NBUF)
                def _():
                    scatter(nxt - NBUF, pb).wait()

                gather(nxt, pb).start()

            gather(j, b).wait()

            @plsc.parallel_loop(0, CH)
            def _(r):
                @plsc.parallel_loop(0, D, step=_LANES, unroll=8)
                def _(c):
                    rows_v[b, r, pl.ds(c, _LANES)] = (
                        rows_v[b, r, pl.ds(c, _LANES)] * jnp.float32(scale)
                    )

            scatter(j, b).start()

        @pl.loop(n_chunks - NBUF, n_chunks)
        def _(j):
            scatter(j, bmod(j)).wait()

    return emb


def kernel(x, table):
    BATCH, SEQ = x.shape
    V, D = table.shape
    scale = float(math.sqrt(D))
    return _build(BATCH, SEQ, D, scale)(x.astype(jnp.int32), table)
